# baseline (speedup 1.0000x reference)
import jax
import jax.numpy as jnp
from jax import lax
from jax.experimental import pallas as pl
from jax.experimental.pallas import tpu as pltpu
from jax.experimental.pallas import tpu_sc as plsc

NC = 2
NS = 16
NW = NC * NS
CHUNK = 128
NB = 10

ROWS_A = 632


def _sc_mesh():
    return plsc.VectorSubcoreMesh(
        core_axis_name="c", subcore_axis_name="s", num_cores=NC, num_subcores=NS
    )


def _make_deg_body(nch):
  def _deg_body(esd_hbm, zeros_hbm, ones_hbm, out_hbm, acc, idx, ones, sem):
    n = acc.shape[0]
    nch_pad = esd_hbm.shape[0]
    c = lax.axis_index("c")
    s = lax.axis_index("s")
    wid = s * NC + c
    rows_b = n - (NS - 1) * ROWS_A
    start = s * ROWS_A

    pltpu.sync_copy(ones_hbm, ones)

    @pl.when(s < NS - 1)
    def _():
        pltpu.sync_copy(zeros_hbm.at[pl.ds(start, ROWS_A)],
                        acc.at[pl.ds(start, ROWS_A)])

    @pl.when(s == NS - 1)
    def _():
        pltpu.sync_copy(zeros_hbm.at[pl.ds(start, rows_b)],
                        acc.at[pl.ds(start, rows_b)])

    plsc.subcore_barrier()

    stride = NW * NB
    nbody = nch_pad // stride

    def body(jj, carry):
        base_c = wid * NB + jj * stride
        pltpu.sync_copy(esd_hbm.at[pl.ds(base_c, NB), 1], idx)

        for b in range(NB):
            @pl.when(base_c + b < nch)
            def _(b=b):
                pltpu.async_copy(ones, acc.at[idx.at[b]], sem, add=True)

        for b in range(NB):
            @pl.when(base_c + b < nch)
            def _(b=b):
                pltpu.make_async_copy(ones, acc.at[idx.at[b]], sem).wait()

        return carry

    lax.fori_loop(0, nbody, body, 0)
    plsc.subcore_barrier()

    @pl.when(s < NS - 1)
    def _():
        pltpu.sync_copy(acc.at[pl.ds(start, ROWS_A)],
                        out_hbm.at[pl.ds(c * n + start, ROWS_A)])

    @pl.when(s == NS - 1)
    def _():
        pltpu.sync_copy(acc.at[pl.ds(start, rows_b)],
                        out_hbm.at[pl.ds(c * n + start, rows_b)])

  return _deg_body


def _make_prop_body(nch):
  def _prop_body(g_hbm, esd_hbm, zeros_hbm, out_hbm,
                 acc, isd, buf, *sems):
    semi = list(sems[:2])
    semg = list(sems[2:2 + NB])
    sems_sc = list(sems[2 + NB:])
    n = g_hbm.shape[0]
    nch_pad = esd_hbm.shape[0]
    c = lax.axis_index("c")
    s = lax.axis_index("s")
    wid = s * NC + c
    rows_b = n - (NS - 1) * ROWS_A
    start = s * ROWS_A

    def seed(src_hbm_ref, rstart, rsize):
        pltpu.sync_copy(src_hbm_ref.at[pl.ds(rstart, rsize)],
                        acc.at[pl.ds(rstart, rsize)])

    @pl.when(jnp.logical_and(s < NS - 1, c == 0))
    def _():
        seed(g_hbm, start, ROWS_A)

    @pl.when(jnp.logical_and(s == NS - 1, c == 0))
    def _():
        seed(g_hbm, start, rows_b)

    @pl.when(jnp.logical_and(s < NS - 1, c != 0))
    def _():
        seed(zeros_hbm, start, ROWS_A)

    @pl.when(jnp.logical_and(s == NS - 1, c != 0))
    def _():
        seed(zeros_hbm, start, rows_b)

    plsc.subcore_barrier()

    stride = NW * NB
    nbody = nch_pad // stride

    def idx_load(base_c, k):
        pltpu.async_copy(esd_hbm.at[pl.ds(base_c, NB)], isd.at[k], semi[k])

    idx_load(wid * NB, 0)

    def half(jj, k):
        base_c = wid * NB + jj * stride
        kn = 1 - k
        prev_c = base_c - stride

        pltpu.make_async_copy(esd_hbm.at[pl.ds(0, NB)], isd.at[k],
                              semi[k]).wait()

        for b in range(NB):
            @pl.when(jnp.logical_and(prev_c >= 0, prev_c + b < nch))
            def _(b=b):
                pltpu.make_async_copy(buf.at[b], acc.at[isd.at[kn, b, 1]],
                                      sems_sc[b]).wait()

            @pl.when(base_c + b < nch)
            def _(b=b):
                pltpu.async_copy(g_hbm.at[isd.at[k, b, 0]],
                                 buf.at[b], semg[b])

        @pl.when(base_c + stride + NB <= nch_pad)
        def _():
            idx_load(base_c + stride, kn)

        for b in range(NB):
            @pl.when(base_c + b < nch)
            def _(b=b):
                pltpu.make_async_copy(g_hbm.at[isd.at[k, b, 0]],
                                      buf.at[b], semg[b]).wait()
                pltpu.async_copy(buf.at[b], acc.at[isd.at[k, b, 1]],
                                 sems_sc[b], add=True)

    def body(jj2, carry):
        half(jj2 * 2, 0)
        half(jj2 * 2 + 1, 1)
        return carry

    lax.fori_loop(0, nbody // 2, body, 0)

    last_c = wid * NB + (nbody - 1) * stride
    k_last = (nbody - 1) % 2
    for b in range(NB):
        @pl.when(last_c + b < nch)
        def _(b=b):
            pltpu.make_async_copy(buf.at[b], acc.at[isd.at[k_last, b, 1]],
                                  sems_sc[b]).wait()

    plsc.subcore_barrier()

    def wback(rstart, rsize):
        pltpu.sync_copy(acc.at[pl.ds(rstart, rsize)],
                        out_hbm.at[pl.ds(rstart, rsize),
                                   pl.ds(c * acc.shape[1], acc.shape[1])])

    @pl.when(s < NS - 1)
    def _():
        wback(start, ROWS_A)

    @pl.when(s == NS - 1)
    def _():
        wback(start, rows_b)

  return _prop_body


def _init_body(d0, d1, x, w, dis_ref, g_ref):
    deg = d0[...] + d1[...] + 1.0
    dis = lax.rsqrt(deg)
    dis_ref[...] = dis
    g_ref[...] = dis * jnp.dot(x[...], w[...],
                               preferred_element_type=jnp.float32)


def _fuse_body(p, dis, b, w, g_ref):
    ch = w.shape[0]
    a = jnp.maximum(
        dis[...] * (p[:, :ch] + p[:, ch:]) + b[...], 0.0)
    g_ref[...] = dis[...] * jnp.dot(a, w[...],
                                    preferred_element_type=jnp.float32)


def _last_body(p, dis, b, a_ref, cs_ref):
    ch = b.shape[1]
    a = jnp.maximum(
        dis[...] * (p[:, :ch] + p[:, ch:]) + b[...], 0.0)
    a_ref[...] = a

    @pl.when(pl.program_id(0) == 0)
    def _():
        cs_ref[...] = jnp.zeros_like(cs_ref)

    cs_ref[...] += jnp.sum(a, axis=0, keepdims=True)


def _head_body(a, cs, wn, wg, wa1, wa2, out_ref):
    pool = jnp.dot(cs[...], wn[...], preferred_element_type=jnp.float32)
    pt = jnp.dot(pool, wg[...], preferred_element_type=jnp.float32)
    s2 = jnp.dot(jnp.maximum(pt, 0.0), wa2[...],
                 preferred_element_type=jnp.float32)
    hn = jnp.dot(a[...], wn[...], preferred_element_type=jnp.float32)
    z1 = jnp.maximum(hn, 0.0)
    out_ref[...] = jnp.tanh(
        jnp.dot(z1, wa1[...], preferred_element_type=jnp.float32) + s2)


def kernel(x, edge_index, W0, Wh, bs, Wnode, Wneig, Waggr):
    n, cin = x.shape
    e = edge_index.shape[1]
    ch = W0.shape[1]
    zeros2d = jnp.zeros((n, ch), jnp.float32)
    zeros1d = jnp.zeros((n,), jnp.float32)
    mesh = _sc_mesh()

    nch = e // CHUNK
    stride = NW * NB
    nbody = (nch + stride - 1) // stride
    nbody += nbody % 2
    nch_pad = nbody * stride
    esd = jnp.transpose(edge_index.reshape(2, nch, CHUNK), (1, 0, 2))
    esd = jnp.concatenate(
        [esd, jnp.zeros((nch_pad - nch, 2, CHUNK), jnp.int32)], axis=0)

    deg_call = pl.kernel(
        _make_deg_body(nch),
        out_type=jax.ShapeDtypeStruct((NC * n,), jnp.float32),
        mesh=mesh,
        compiler_params=pltpu.CompilerParams(use_tc_tiling_on_sc=False),
        scratch_types=[
            pltpu.VMEM_SHARED((n,), jnp.float32),
            pltpu.VMEM((NB, CHUNK), jnp.int32),
            pltpu.VMEM((CHUNK,), jnp.float32),
            pltpu.SemaphoreType.DMA,
        ],
    )
    deg_flat = deg_call(esd, zeros1d, jnp.ones((CHUNK,), jnp.float32))
    d0 = deg_flat[:n].reshape(n, 1)
    d1 = deg_flat[n:].reshape(n, 1)

    prop_call = pl.kernel(
        _make_prop_body(nch),
        out_type=jax.ShapeDtypeStruct((n, NC * ch), jnp.float32),
        mesh=mesh,
        compiler_params=pltpu.CompilerParams(use_tc_tiling_on_sc=False),
        scratch_types=[
            pltpu.VMEM_SHARED((n, ch), jnp.float32),
            pltpu.VMEM((2, NB, 2, CHUNK), jnp.int32),
            pltpu.VMEM((NB, CHUNK, ch), jnp.float32),
        ] + [pltpu.SemaphoreType.DMA] * (2 + 2 * NB),
    )

    R = 5000
    grid = (n // R,)
    nblk = n // R
    row_spec = pl.BlockSpec((R, ch), lambda i: (i, 0))
    p1_spec = pl.BlockSpec((R, ch), lambda i: (i + nblk, 0))
    col_spec = pl.BlockSpec((R, 1), lambda i: (i, 0))
    w_spec = pl.BlockSpec((ch, ch), lambda i: (0, 0))
    b_spec = pl.BlockSpec((1, ch), lambda i: (0, 0))

    dis, g = pl.pallas_call(
        _init_body,
        grid=grid,
        in_specs=[
            col_spec, col_spec,
            pl.BlockSpec((R, cin), lambda i: (i, 0)),
            pl.BlockSpec((cin, ch), lambda i: (0, 0)),
        ],
        out_specs=[col_spec, row_spec],
        out_shape=[
            jax.ShapeDtypeStruct((n, 1), jnp.float32),
            jax.ShapeDtypeStruct((n, ch), jnp.float32),
        ],
    )(d0, d1, x, W0)

    p_spec = pl.BlockSpec((R, NC * ch), lambda i: (i, 0))
    fuse_call = pl.pallas_call(
        _fuse_body,
        grid=grid,
        in_specs=[p_spec, col_spec, b_spec, w_spec],
        out_specs=row_spec,
        out_shape=jax.ShapeDtypeStruct((n, ch), jnp.float32),
    )

    num_layers = Wh.shape[0] + 1
    for i in range(num_layers):
        p = prop_call(g, esd, zeros2d)
        b_i = bs[i].reshape(1, ch)
        if i < num_layers - 1:
            g = fuse_call(p, dis, b_i, Wh[i])
        else:
            a9, cs = pl.pallas_call(
                _last_body,
                grid=grid,
                in_specs=[p_spec, col_spec, b_spec],
                out_specs=[row_spec, pl.BlockSpec((1, ch), lambda i: (0, 0))],
                out_shape=[
                    jax.ShapeDtypeStruct((n, ch), jnp.float32),
                    jax.ShapeDtypeStruct((1, ch), jnp.float32),
                ],
            )(p, dis, b_i)

    out = pl.pallas_call(
        _head_body,
        grid=grid,
        in_specs=[
            row_spec,
            pl.BlockSpec((1, ch), lambda i: (0, 0)),
            w_spec, w_spec,
            pl.BlockSpec((ch, 1), lambda i: (0, 0)),
            pl.BlockSpec((ch, 1), lambda i: (0, 0)),
        ],
        out_specs=pl.BlockSpec((R, 1), lambda i: (i, 0)),
        out_shape=jax.ShapeDtypeStruct((n, 1), jnp.float32),
    )(a9, cs, Wnode, Wneig, Waggr[:ch], Waggr[ch:])
    return out


# speedup vs baseline: 1.0113x; 1.0113x over previous
import jax
import jax.numpy as jnp
from jax import lax
from jax.experimental import pallas as pl
from jax.experimental.pallas import tpu as pltpu
from jax.experimental.pallas import tpu_sc as plsc

NC = 2
NS = 16
NW = NC * NS
CHUNK = 128
NB = 9

ROWS_A = 632


def _sc_mesh():
    return plsc.VectorSubcoreMesh(
        core_axis_name="c", subcore_axis_name="s", num_cores=NC, num_subcores=NS
    )


def _make_deg_body(nch):
  def _deg_body(esd_hbm, zeros_hbm, ones_hbm, out_hbm, acc, idx, ones, sem):
    n = acc.shape[0]
    nch_pad = esd_hbm.shape[0]
    c = lax.axis_index("c")
    s = lax.axis_index("s")
    wid = s * NC + c
    rows_b = n - (NS - 1) * ROWS_A
    start = s * ROWS_A

    pltpu.sync_copy(ones_hbm, ones)

    @pl.when(s < NS - 1)
    def _():
        pltpu.sync_copy(zeros_hbm.at[pl.ds(start, ROWS_A)],
                        acc.at[pl.ds(start, ROWS_A)])

    @pl.when(s == NS - 1)
    def _():
        pltpu.sync_copy(zeros_hbm.at[pl.ds(start, rows_b)],
                        acc.at[pl.ds(start, rows_b)])

    plsc.subcore_barrier()

    stride = NW * NB
    nbody = nch_pad // stride

    def body(jj, carry):
        base_c = wid * NB + jj * stride
        pltpu.sync_copy(esd_hbm.at[pl.ds(base_c, NB), 1], idx)

        for b in range(NB):
            @pl.when(base_c + b < nch)
            def _(b=b):
                pltpu.async_copy(ones, acc.at[idx.at[b]], sem, add=True)

        for b in range(NB):
            @pl.when(base_c + b < nch)
            def _(b=b):
                pltpu.make_async_copy(ones, acc.at[idx.at[b]], sem).wait()

        return carry

    lax.fori_loop(0, nbody, body, 0)
    plsc.subcore_barrier()

    @pl.when(s < NS - 1)
    def _():
        pltpu.sync_copy(acc.at[pl.ds(start, ROWS_A)],
                        out_hbm.at[pl.ds(c * n + start, ROWS_A)])

    @pl.when(s == NS - 1)
    def _():
        pltpu.sync_copy(acc.at[pl.ds(start, rows_b)],
                        out_hbm.at[pl.ds(c * n + start, rows_b)])

  return _deg_body


def _make_prop_body(nch):
  def _prop_body(g_hbm, esd_hbm, zeros_hbm, out_hbm,
                 acc, isd, buf, *sems):
    semi = list(sems[:2])
    semg = list(sems[2:2 + NB])
    sems_sc = list(sems[2 + NB:])
    n = g_hbm.shape[0]
    nch_pad = esd_hbm.shape[0]
    c = lax.axis_index("c")
    s = lax.axis_index("s")
    wid = s * NC + c
    rows_b = n - (NS - 1) * ROWS_A
    start = s * ROWS_A

    def seed(src_hbm_ref, rstart, rsize):
        pltpu.sync_copy(src_hbm_ref.at[pl.ds(rstart, rsize)],
                        acc.at[pl.ds(rstart, rsize)])

    @pl.when(jnp.logical_and(s < NS - 1, c == 0))
    def _():
        seed(g_hbm, start, ROWS_A)

    @pl.when(jnp.logical_and(s == NS - 1, c == 0))
    def _():
        seed(g_hbm, start, rows_b)

    @pl.when(jnp.logical_and(s < NS - 1, c != 0))
    def _():
        seed(zeros_hbm, start, ROWS_A)

    @pl.when(jnp.logical_and(s == NS - 1, c != 0))
    def _():
        seed(zeros_hbm, start, rows_b)

    plsc.subcore_barrier()

    stride = NW * NB
    nbody = nch_pad // stride

    def idx_load(base_c, k):
        pltpu.async_copy(esd_hbm.at[pl.ds(base_c, NB)], isd.at[k], semi[k])

    idx_load(wid * NB, 0)

    def half(jj, k):
        base_c = wid * NB + jj * stride
        kn = 1 - k
        prev_c = base_c - stride

        pltpu.make_async_copy(esd_hbm.at[pl.ds(0, NB)], isd.at[k],
                              semi[k]).wait()

        for b in range(NB):
            @pl.when(jnp.logical_and(prev_c >= 0, prev_c + b < nch))
            def _(b=b):
                pltpu.make_async_copy(buf.at[b], acc.at[isd.at[kn, b, 1]],
                                      sems_sc[b]).wait()

            @pl.when(base_c + b < nch)
            def _(b=b):
                pltpu.async_copy(g_hbm.at[isd.at[k, b, 0]],
                                 buf.at[b], semg[b])

        @pl.when(base_c + stride + NB <= nch_pad)
        def _():
            idx_load(base_c + stride, kn)

        for b in range(NB):
            @pl.when(base_c + b < nch)
            def _(b=b):
                pltpu.make_async_copy(g_hbm.at[isd.at[k, b, 0]],
                                      buf.at[b], semg[b]).wait()
                pltpu.async_copy(buf.at[b], acc.at[isd.at[k, b, 1]],
                                 sems_sc[b], add=True)

    def body(jj2, carry):
        half(jj2 * 2, 0)
        half(jj2 * 2 + 1, 1)
        return carry

    lax.fori_loop(0, nbody // 2, body, 0)

    last_c = wid * NB + (nbody - 1) * stride
    k_last = (nbody - 1) % 2
    for b in range(NB):
        @pl.when(last_c + b < nch)
        def _(b=b):
            pltpu.make_async_copy(buf.at[b], acc.at[isd.at[k_last, b, 1]],
                                  sems_sc[b]).wait()

    plsc.subcore_barrier()

    def wback(rstart, rsize):
        pltpu.sync_copy(acc.at[pl.ds(rstart, rsize)],
                        out_hbm.at[pl.ds(rstart, rsize),
                                   pl.ds(c * acc.shape[1], acc.shape[1])])

    @pl.when(s < NS - 1)
    def _():
        wback(start, ROWS_A)

    @pl.when(s == NS - 1)
    def _():
        wback(start, rows_b)

  return _prop_body


def _init_body(d0, d1, x, w, dis_ref, g_ref):
    deg = d0[...] + d1[...] + 1.0
    dis = lax.rsqrt(deg)
    dis_ref[...] = dis
    g_ref[...] = dis * jnp.dot(x[...], w[...],
                               preferred_element_type=jnp.float32)


def _fuse_body(p, dis, b, w, g_ref):
    ch = w.shape[0]
    a = jnp.maximum(
        dis[...] * (p[:, :ch] + p[:, ch:]) + b[...], 0.0)
    g_ref[...] = dis[...] * jnp.dot(a, w[...],
                                    preferred_element_type=jnp.float32)


def _last_body(p, dis, b, a_ref, cs_ref):
    ch = b.shape[1]
    a = jnp.maximum(
        dis[...] * (p[:, :ch] + p[:, ch:]) + b[...], 0.0)
    a_ref[...] = a

    @pl.when(pl.program_id(0) == 0)
    def _():
        cs_ref[...] = jnp.zeros_like(cs_ref)

    cs_ref[...] += jnp.sum(a, axis=0, keepdims=True)


def _head_body(a, cs, wn, wg, wa1, wa2, out_ref):
    pool = jnp.dot(cs[...], wn[...], preferred_element_type=jnp.float32)
    pt = jnp.dot(pool, wg[...], preferred_element_type=jnp.float32)
    s2 = jnp.dot(jnp.maximum(pt, 0.0), wa2[...],
                 preferred_element_type=jnp.float32)
    hn = jnp.dot(a[...], wn[...], preferred_element_type=jnp.float32)
    z1 = jnp.maximum(hn, 0.0)
    out_ref[...] = jnp.tanh(
        jnp.dot(z1, wa1[...], preferred_element_type=jnp.float32) + s2)


def kernel(x, edge_index, W0, Wh, bs, Wnode, Wneig, Waggr):
    n, cin = x.shape
    e = edge_index.shape[1]
    ch = W0.shape[1]
    zeros2d = jnp.zeros((n, ch), jnp.float32)
    zeros1d = jnp.zeros((n,), jnp.float32)
    mesh = _sc_mesh()

    nch = e // CHUNK
    stride = NW * NB
    nbody = (nch + stride - 1) // stride
    nbody += nbody % 2
    nch_pad = nbody * stride
    esd = jnp.transpose(edge_index.reshape(2, nch, CHUNK), (1, 0, 2))
    esd = jnp.concatenate(
        [esd, jnp.zeros((nch_pad - nch, 2, CHUNK), jnp.int32)], axis=0)

    deg_call = pl.kernel(
        _make_deg_body(nch),
        out_type=jax.ShapeDtypeStruct((NC * n,), jnp.float32),
        mesh=mesh,
        compiler_params=pltpu.CompilerParams(use_tc_tiling_on_sc=False),
        scratch_types=[
            pltpu.VMEM_SHARED((n,), jnp.float32),
            pltpu.VMEM((NB, CHUNK), jnp.int32),
            pltpu.VMEM((CHUNK,), jnp.float32),
            pltpu.SemaphoreType.DMA,
        ],
    )
    deg_flat = deg_call(esd, zeros1d, jnp.ones((CHUNK,), jnp.float32))
    d0 = deg_flat[:n].reshape(n, 1)
    d1 = deg_flat[n:].reshape(n, 1)

    prop_call = pl.kernel(
        _make_prop_body(nch),
        out_type=jax.ShapeDtypeStruct((n, NC * ch), jnp.float32),
        mesh=mesh,
        compiler_params=pltpu.CompilerParams(use_tc_tiling_on_sc=False),
        scratch_types=[
            pltpu.VMEM_SHARED((n, ch), jnp.float32),
            pltpu.VMEM((2, NB, 2, CHUNK), jnp.int32),
            pltpu.VMEM((NB, CHUNK, ch), jnp.float32),
        ] + [pltpu.SemaphoreType.DMA] * (2 + 2 * NB),
    )

    R = 5000
    grid = (n // R,)
    nblk = n // R
    row_spec = pl.BlockSpec((R, ch), lambda i: (i, 0))
    p1_spec = pl.BlockSpec((R, ch), lambda i: (i + nblk, 0))
    col_spec = pl.BlockSpec((R, 1), lambda i: (i, 0))
    w_spec = pl.BlockSpec((ch, ch), lambda i: (0, 0))
    b_spec = pl.BlockSpec((1, ch), lambda i: (0, 0))

    dis, g = pl.pallas_call(
        _init_body,
        grid=grid,
        in_specs=[
            col_spec, col_spec,
            pl.BlockSpec((R, cin), lambda i: (i, 0)),
            pl.BlockSpec((cin, ch), lambda i: (0, 0)),
        ],
        out_specs=[col_spec, row_spec],
        out_shape=[
            jax.ShapeDtypeStruct((n, 1), jnp.float32),
            jax.ShapeDtypeStruct((n, ch), jnp.float32),
        ],
    )(d0, d1, x, W0)

    p_spec = pl.BlockSpec((R, NC * ch), lambda i: (i, 0))
    fuse_call = pl.pallas_call(
        _fuse_body,
        grid=grid,
        in_specs=[p_spec, col_spec, b_spec, w_spec],
        out_specs=row_spec,
        out_shape=jax.ShapeDtypeStruct((n, ch), jnp.float32),
    )

    num_layers = Wh.shape[0] + 1
    for i in range(num_layers):
        p = prop_call(g, esd, zeros2d)
        b_i = bs[i].reshape(1, ch)
        if i < num_layers - 1:
            g = fuse_call(p, dis, b_i, Wh[i])
        else:
            a9, cs = pl.pallas_call(
                _last_body,
                grid=grid,
                in_specs=[p_spec, col_spec, b_spec],
                out_specs=[row_spec, pl.BlockSpec((1, ch), lambda i: (0, 0))],
                out_shape=[
                    jax.ShapeDtypeStruct((n, ch), jnp.float32),
                    jax.ShapeDtypeStruct((1, ch), jnp.float32),
                ],
            )(p, dis, b_i)

    out = pl.pallas_call(
        _head_body,
        grid=grid,
        in_specs=[
            row_spec,
            pl.BlockSpec((1, ch), lambda i: (0, 0)),
            w_spec, w_spec,
            pl.BlockSpec((ch, 1), lambda i: (0, 0)),
            pl.BlockSpec((ch, 1), lambda i: (0, 0)),
        ],
        out_specs=pl.BlockSpec((R, 1), lambda i: (i, 0)),
        out_shape=jax.ShapeDtypeStruct((n, 1), jnp.float32),
    )(a9, cs, Wnode, Wneig, Waggr[:ch], Waggr[ch:])
    return out
